# baseline (speedup 1.0000x reference)
import functools

import jax
import jax.numpy as jnp
from jax import lax
from jax.experimental import pallas as pl
from jax.experimental.pallas import tpu as pltpu
from jax.experimental.pallas import tpu_sc as plsc

B = 4096
S = 50
D = 768
H = 768
VOCAB = 50257
K_TOT = S * D

NC = 2
NS = 16
NW = NC * NS
N_IDX = B * S
B_PER_W = N_IDX // NW
CHUNK = 64
NCHUNK = B_PER_W // CHUNK


def _sc_gather(table, idx):
    mesh = plsc.VectorSubcoreMesh(core_axis_name="c", subcore_axis_name="s")
    d_words = table.shape[1]

    @functools.partial(
        pl.kernel,
        mesh=mesh,
        out_type=jax.ShapeDtypeStruct((N_IDX, d_words), table.dtype),
        scratch_types=[
            pltpu.VMEM((2, CHUNK), jnp.int32),
            pltpu.VMEM((2, CHUNK, d_words), table.dtype),
            pltpu.SemaphoreType.DMA,
            pltpu.SemaphoreType.DMA,
        ],
    )
    def k(table_hbm, idx_hbm, out_hbm, idx_c, rows_v, sem0, sem1):
        wid = lax.axis_index("s") * NC + lax.axis_index("c")
        base = wid * B_PER_W
        sems = (sem0, sem1)

        def start(c, slot):
            pltpu.sync_copy(idx_hbm.at[pl.ds(base + c * CHUNK, CHUNK)],
                            idx_c.at[slot])
            pltpu.async_copy(table_hbm.at[idx_c.at[slot]], rows_v.at[slot],
                             sems[slot])

        def wait(slot):
            pltpu.make_async_copy(table_hbm.at[idx_c.at[slot]],
                                  rows_v.at[slot], sems[slot]).wait()

        for s in range(2):
            start(s, s)

        @pl.loop(0, NCHUNK // 2)
        def _pair(p):
            for s in range(2):
                c = p * 2 + s
                wait(s)
                pltpu.sync_copy(
                    rows_v.at[s],
                    out_hbm.at[pl.ds(base + c * CHUNK, CHUNK)])
                nxt = c + 2

                @pl.when(nxt < NCHUNK)
                def _():
                    start(nxt, s)

    return k(table, idx)


M_BLK = 2048
K_BLK = 1920


def _mm_body(h_ref, w_ref, b_ref, o_ref, acc_ref):
    kstep = pl.program_id(1)

    @pl.when(kstep == 0)
    def _():
        acc_ref[...] = jnp.broadcast_to(b_ref[...], acc_ref.shape)

    acc_ref[...] += lax.dot_general(
        h_ref[...], w_ref[...], (((1,), (1,)), ((), ())),
        preferred_element_type=jnp.float32)

    @pl.when(kstep == pl.num_programs(1) - 1)
    def _():
        o_ref[...] = acc_ref[...]


def _tc_matmul(h, W, b):
    grid = (B // M_BLK, K_TOT // K_BLK)
    return pl.pallas_call(
        _mm_body,
        grid=grid,
        in_specs=[
            pl.BlockSpec((M_BLK, K_BLK), lambda m, k: (m, k)),
            pl.BlockSpec((H, K_BLK), lambda m, k: (0, k)),
            pl.BlockSpec((1, H), lambda m, k: (0, 0)),
        ],
        out_specs=pl.BlockSpec((M_BLK, H), lambda m, k: (m, 0)),
        out_shape=jax.ShapeDtypeStruct((B, H), jnp.float32),
        scratch_shapes=[pltpu.VMEM((M_BLK, H), jnp.float32)],
    )(h, W, b.reshape(1, H))


def kernel(x, emb_table, W, b):
    idx = x.reshape(-1)
    t16 = emb_table.astype(jnp.bfloat16)
    t_i32 = lax.bitcast_convert_type(t16.reshape(VOCAB, D // 2, 2),
                                     jnp.int32)
    g = _sc_gather(t_i32, idx)
    h = lax.bitcast_convert_type(g, jnp.bfloat16).reshape(B, K_TOT)
    return _tc_matmul(h, W.astype(jnp.bfloat16), b)


# speedup vs baseline: 37.2229x; 37.2229x over previous
import functools

import jax
import jax.numpy as jnp
from jax import lax
from jax.experimental import pallas as pl
from jax.experimental.pallas import tpu as pltpu
from jax.experimental.pallas import tpu_sc as plsc

B = 4096
S = 50
D = 768
H = 768
VOCAB = 50257
K_TOT = S * D

NC = 2
NS = 16
NW = NC * NS
N_IDX = B * S
B_PER_W = N_IDX // NW
CHUNK = 64
NCHUNK = B_PER_W // CHUNK


def _sc_gather(table, idx):
    mesh = plsc.VectorSubcoreMesh(core_axis_name="c", subcore_axis_name="s")
    d_words = table.shape[1]
    n_idx = idx.shape[0]
    b_per_w = n_idx // NW
    nchunk = b_per_w // CHUNK

    @functools.partial(
        pl.kernel,
        mesh=mesh,
        out_type=jax.ShapeDtypeStruct((n_idx, d_words), table.dtype),
        scratch_types=[
            pltpu.VMEM((2, CHUNK), jnp.int32),
            pltpu.VMEM((2, CHUNK, d_words), table.dtype),
            pltpu.SemaphoreType.DMA,
            pltpu.SemaphoreType.DMA,
        ],
    )
    def k(table_hbm, idx_hbm, out_hbm, idx_c, rows_v, sem0, sem1):
        wid = lax.axis_index("s") * NC + lax.axis_index("c")
        base = wid * b_per_w
        sems = (sem0, sem1)

        def start(c, slot):
            pltpu.sync_copy(idx_hbm.at[pl.ds(base + c * CHUNK, CHUNK)],
                            idx_c.at[slot])
            pltpu.async_copy(table_hbm.at[idx_c.at[slot]], rows_v.at[slot],
                             sems[slot])

        def wait(slot):
            pltpu.make_async_copy(table_hbm.at[idx_c.at[slot]],
                                  rows_v.at[slot], sems[slot]).wait()

        for s in range(2):
            start(s, s)

        @pl.loop(0, nchunk // 2)
        def _pair(p):
            for s in range(2):
                c = p * 2 + s
                wait(s)
                pltpu.sync_copy(
                    rows_v.at[s],
                    out_hbm.at[pl.ds(base + c * CHUNK, CHUNK)])
                nxt = c + 2

                @pl.when(nxt < nchunk)
                def _():
                    start(nxt, s)

    return k(table, idx)


M_BLK = 2048
K_BLK = 1280


def _mm_body(h_ref, w_ref, b_ref, o_ref, acc_ref):
    kstep = pl.program_id(1)

    @pl.when(kstep == 0)
    def _():
        acc_ref[...] = jnp.broadcast_to(b_ref[...], acc_ref.shape)

    acc_ref[...] += lax.dot_general(
        h_ref[...].astype(jnp.bfloat16), w_ref[...],
        (((1,), (1,)), ((), ())),
        preferred_element_type=jnp.float32)

    @pl.when(kstep == pl.num_programs(1) - 1)
    def _():
        o_ref[...] = acc_ref[...]


def _tc_matmul(h, W, b):
    m_tot = h.shape[0]
    grid = (m_tot // M_BLK, K_TOT // K_BLK)
    return pl.pallas_call(
        _mm_body,
        grid=grid,
        in_specs=[
            pl.BlockSpec((M_BLK, K_BLK), lambda m, k: (m, k)),
            pl.BlockSpec((H, K_BLK), lambda m, k: (0, k)),
            pl.BlockSpec((1, H), lambda m, k: (0, 0)),
        ],
        out_specs=pl.BlockSpec((M_BLK, H), lambda m, k: (m, 0)),
        out_shape=jax.ShapeDtypeStruct((m_tot, H), jnp.float32),
        scratch_shapes=[pltpu.VMEM((M_BLK, H), jnp.float32)],
    )(h, W, b.reshape(1, H))


NCH = 2
BCH = B // NCH


def kernel(x, emb_table, W, b):
    w16 = W.astype(jnp.bfloat16)
    outs = []
    for c in range(NCH):
        idx_c = x[c * BCH:(c + 1) * BCH].reshape(-1)
        g = _sc_gather(emb_table, idx_c)
        h = g.reshape(BCH, K_TOT)
        outs.append(_tc_matmul(h, w16, b))
    return jnp.concatenate(outs, axis=0)
